# baseline (speedup 1.0000x reference)
import functools

import jax
import jax.numpy as jnp
from jax import lax
from jax.experimental import pallas as pl
from jax.experimental.pallas import tpu as pltpu
from jax.experimental.pallas import tpu_sc as plsc

N = 320000
D = 128
S = 512
NC = 2
NS = 16
NW = NC * NS
CHUNK = 256
NCHUNK = N // CHUNK
NBASE = NCHUNK // NW
NEXTRA = NCHUNK - NBASE * NW

assert N % CHUNK == 0 and CHUNK % 128 == 0


def _sc_body(x_hbm, batch_hbm, sums_hbm, cnts_hbm,
             x_bufs, idx_lo, idx_hi, ones_v, zero_v,
             acc_sh, cnt_sh, semx0, semx1, semi0, semi1):
    cid = lax.axis_index("c")
    sid = lax.axis_index("s")
    wid = cid * NS + sid
    semx = (semx0, semx1)
    semi = (semi0, semi1)

    zeros16 = jnp.zeros((16,), jnp.float32)
    ones16 = jnp.ones((16,), jnp.float32)
    for j in range(CHUNK // 16):
        ones_v[pl.ds(j * 16, 16)] = ones16
    for i in range(S // NS):
        for j in range(D // 16):
            zero_v[i, pl.ds(j * 16, 16)] = zeros16

    pltpu.sync_copy(zero_v, acc_sh.at[pl.ds(sid * (S // NS), S // NS)])
    pltpu.sync_copy(zero_v.at[0, pl.ds(0, S // NS)],
                    cnt_sh.at[pl.ds(sid * (S // NS), S // NS)])
    plsc.subcore_barrier()

    def start_load(c, b):
        row = c * CHUNK
        pltpu.async_copy(x_hbm.at[pl.ds(row, CHUNK), :], x_bufs.at[b], semx[b])
        pltpu.async_copy(batch_hbm.at[pl.ds(row, 128)], idx_lo.at[b], semi[b])
        pltpu.async_copy(batch_hbm.at[pl.ds(row + 128, 128)], idx_hi.at[b],
                         semi[b])

    def wait_load(c, b):
        row = c * CHUNK
        pltpu.make_async_copy(x_hbm.at[pl.ds(row, CHUNK), :], x_bufs.at[b],
                              semx[b]).wait()
        pltpu.make_async_copy(batch_hbm.at[pl.ds(row, 128)], idx_lo.at[b],
                              semi[b]).wait()
        pltpu.make_async_copy(batch_hbm.at[pl.ds(row + 128, 128)], idx_hi.at[b],
                              semi[b]).wait()

    def scatter(b):
        pltpu.sync_copy(x_bufs.at[b, pl.ds(0, 128)],
                        acc_sh.at[idx_lo.at[b]], add=True)
        pltpu.sync_copy(x_bufs.at[b, pl.ds(128, 128)],
                        acc_sh.at[idx_hi.at[b]], add=True)
        pltpu.sync_copy(ones_v.at[pl.ds(0, 128)],
                        cnt_sh.at[idx_lo.at[b]], add=True)
        pltpu.sync_copy(ones_v.at[pl.ds(0, 128)],
                        cnt_sh.at[idx_hi.at[b]], add=True)

    start_load(wid, 0)
    start_load(wid + NW, 1)

    def body(tt, carry):
        for b in range(2):
            t = tt * 2 + b
            c = wid + t * NW
            wait_load(c, b)
            scatter(b)

            @pl.when(t + 2 < NBASE)
            def _():
                start_load(wid + (t + 2) * NW, b)
        return carry

    lax.fori_loop(0, NBASE // 2, body, 0)

    if NBASE % 2:
        t = NBASE - 1
        c = wid + t * NW
        wait_load(c, t % 2)
        scatter(t % 2)

    @pl.when(wid < NEXTRA)
    def _():
        c = NBASE * NW + wid
        start_load(c, 0)
        wait_load(c, 0)
        scatter(0)

    plsc.subcore_barrier()

    r0 = sid * (S // NS)
    pltpu.sync_copy(acc_sh.at[pl.ds(r0, S // NS)],
                    sums_hbm.at[cid, pl.ds(r0, S // NS)])
    cnt_v = ones_v.at[pl.ds(0, S // NS)]
    pltpu.sync_copy(cnt_sh.at[pl.ds(r0, S // NS)], cnt_v)
    pltpu.sync_copy(cnt_v, cnts_hbm.at[cid, pl.ds(r0, S // NS)])


_sc_segment_sum = functools.partial(
    pl.kernel,
    out_type=[
        jax.ShapeDtypeStruct((NC, S, D), jnp.float32),
        jax.ShapeDtypeStruct((NC, S), jnp.float32),
    ],
    mesh=plsc.VectorSubcoreMesh(core_axis_name="c", subcore_axis_name="s"),
    scratch_types=[
        pltpu.VMEM((2, CHUNK, D), jnp.float32),
        pltpu.VMEM((2, 128), jnp.int32),
        pltpu.VMEM((2, 128), jnp.int32),
        pltpu.VMEM((CHUNK,), jnp.float32),
        pltpu.VMEM((S // NS, D), jnp.float32),
        pltpu.VMEM_SHARED((S, D), jnp.float32),
        pltpu.VMEM_SHARED((S,), jnp.float32),
        pltpu.SemaphoreType.DMA,
        pltpu.SemaphoreType.DMA,
        pltpu.SemaphoreType.DMA,
        pltpu.SemaphoreType.DMA,
    ],
)(_sc_body)


def _finalize_body(s_ref, c_ref, o_ref):
    s = s_ref[0] + s_ref[1]
    c = jnp.maximum(c_ref[0] + c_ref[1], 1.0)
    o_ref[...] = s / c


_finalize = pl.pallas_call(
    _finalize_body,
    out_shape=jax.ShapeDtypeStruct((S, D), jnp.float32),
)


@jax.jit
def kernel(x, batch):
    sums, cnts = _sc_segment_sum(x, batch.astype(jnp.int32))
    return _finalize(sums, cnts.reshape(NC, S, 1))


# speedup vs baseline: 1.1081x; 1.1081x over previous
import functools

import jax
import jax.numpy as jnp
from jax import lax
from jax.experimental import pallas as pl
from jax.experimental.pallas import tpu as pltpu
from jax.experimental.pallas import tpu_sc as plsc

N = 320000
D = 128
S = 512
NC = 2
NS = 16
NW = NC * NS
CHUNK = 128
NB = 4
NCHUNK = N // CHUNK
NBASE = NCHUNK // NW
NEXTRA = NCHUNK - NBASE * NW

assert N % CHUNK == 0 and NBASE % NB == 2


def _sc_body(x_hbm, batch_hbm, sums_hbm, cnts_hbm,
             x_bufs, idx_bufs, ones_v, zero_v, acc_sh, cnt_sh,
             semx0, semx1, semx2, semx3,
             semi0, semi1, semi2, semi3,
             sems0, sems1, sems2, sems3):
    cid = lax.axis_index("c")
    sid = lax.axis_index("s")
    wid = cid * NS + sid
    semx = (semx0, semx1, semx2, semx3)
    semi = (semi0, semi1, semi2, semi3)
    sems = (sems0, sems1, sems2, sems3)

    zeros16 = jnp.zeros((16,), jnp.float32)
    ones16 = jnp.ones((16,), jnp.float32)
    for j in range(CHUNK // 16):
        ones_v[pl.ds(j * 16, 16)] = ones16
    for i in range(S // NS):
        for j in range(D // 16):
            zero_v[i, pl.ds(j * 16, 16)] = zeros16

    pltpu.sync_copy(zero_v, acc_sh.at[pl.ds(sid * (S // NS), S // NS)])
    pltpu.sync_copy(zero_v.at[0, pl.ds(0, S // NS)],
                    cnt_sh.at[pl.ds(sid * (S // NS), S // NS)])
    plsc.subcore_barrier()

    def chunk_row(t):
        return (wid + t * NW) * CHUNK

    def start_load(t, b):
        row = chunk_row(t)
        pltpu.async_copy(x_hbm.at[pl.ds(row, CHUNK), :], x_bufs.at[b], semx[b])
        pltpu.async_copy(batch_hbm.at[pl.ds(row, CHUNK)], idx_bufs.at[b], semi[b])

    def wait_load(t, b):
        row = chunk_row(t)
        pltpu.make_async_copy(x_hbm.at[pl.ds(row, CHUNK), :], x_bufs.at[b],
                              semx[b]).wait()
        pltpu.make_async_copy(batch_hbm.at[pl.ds(row, CHUNK)], idx_bufs.at[b],
                              semi[b]).wait()

    def start_scatter(b):
        pltpu.async_copy(x_bufs.at[b], acc_sh.at[idx_bufs.at[b]], sems[b],
                         add=True)
        pltpu.async_copy(ones_v, cnt_sh.at[idx_bufs.at[b]], sems[b], add=True)

    def wait_scatter(b):
        pltpu.make_async_copy(x_bufs.at[b], acc_sh.at[idx_bufs.at[b]],
                              sems[b]).wait()
        pltpu.make_async_copy(ones_v, cnt_sh.at[idx_bufs.at[b]],
                              sems[b]).wait()

    start_load(0, 0)
    start_load(1, 1)

    def body(tt, carry):
        for u in range(NB):
            t = tt * NB + u
            b = u
            wait_load(t, b)
            start_scatter(b)
            tf = t + 2
            bf = (u + 2) % NB

            @pl.when(tf < NBASE)
            def _():
                @pl.when(tf >= NB)
                def _():
                    wait_scatter(bf)
                start_load(tf, bf)
        return carry

    lax.fori_loop(0, NBASE // NB, body, 0)

    for t in (NBASE - 2, NBASE - 1):
        b = t % NB
        wait_load(t, b)
        start_scatter(b)

    @pl.when(wid < NEXTRA)
    def _():
        wait_scatter(2)
        row = (NBASE * NW + wid) * CHUNK
        pltpu.async_copy(x_hbm.at[pl.ds(row, CHUNK), :], x_bufs.at[2], semx[2])
        pltpu.async_copy(batch_hbm.at[pl.ds(row, CHUNK)], idx_bufs.at[2],
                         semi[2])
        pltpu.make_async_copy(x_hbm.at[pl.ds(row, CHUNK), :], x_bufs.at[2],
                              semx[2]).wait()
        pltpu.make_async_copy(batch_hbm.at[pl.ds(row, CHUNK)], idx_bufs.at[2],
                              semi[2]).wait()
        start_scatter(2)

    for b in (3, 0, 1, 2):
        wait_scatter(b)

    plsc.subcore_barrier()

    r0 = sid * (S // NS)
    pltpu.sync_copy(acc_sh.at[pl.ds(r0, S // NS)],
                    sums_hbm.at[cid, pl.ds(r0, S // NS)])
    cnt_v = ones_v.at[pl.ds(0, S // NS)]
    pltpu.sync_copy(cnt_sh.at[pl.ds(r0, S // NS)], cnt_v)
    pltpu.sync_copy(cnt_v, cnts_hbm.at[cid, pl.ds(r0, S // NS)])


_sc_segment_sum = functools.partial(
    pl.kernel,
    out_type=[
        jax.ShapeDtypeStruct((NC, S, D), jnp.float32),
        jax.ShapeDtypeStruct((NC, S), jnp.float32),
    ],
    mesh=plsc.VectorSubcoreMesh(core_axis_name="c", subcore_axis_name="s"),
    scratch_types=[
        pltpu.VMEM((NB, CHUNK, D), jnp.float32),
        pltpu.VMEM((NB, CHUNK), jnp.int32),
        pltpu.VMEM((CHUNK,), jnp.float32),
        pltpu.VMEM((S // NS, D), jnp.float32),
        pltpu.VMEM_SHARED((S, D), jnp.float32),
        pltpu.VMEM_SHARED((S,), jnp.float32),
    ] + [pltpu.SemaphoreType.DMA] * 12,
)(_sc_body)


def _finalize_body(s_ref, c_ref, o_ref):
    s = s_ref[0] + s_ref[1]
    c = jnp.maximum(c_ref[0] + c_ref[1], 1.0)
    o_ref[...] = s / c


_finalize = pl.pallas_call(
    _finalize_body,
    out_shape=jax.ShapeDtypeStruct((S, D), jnp.float32),
)


@jax.jit
def kernel(x, batch):
    sums, cnts = _sc_segment_sum(x, batch.astype(jnp.int32))
    return _finalize(sums, cnts.reshape(NC, S, 1))


# speedup vs baseline: 1.1293x; 1.0191x over previous
import functools

import jax
import jax.numpy as jnp
from jax import lax
from jax.experimental import pallas as pl
from jax.experimental.pallas import tpu as pltpu
from jax.experimental.pallas import tpu_sc as plsc

N = 320000
D = 128
S = 512
NC = 2
NS = 16
NW = NC * NS
CHUNK = 128
NB = 6
NCHUNK = N // CHUNK
NBASE = NCHUNK // NW
NEXTRA = NCHUNK - NBASE * NW

assert N % CHUNK == 0 and NBASE % NB == 0


def _sc_body(x_hbm, batch_hbm, sums_hbm, cnts_hbm,
             x_bufs, idx_bufs, ones_v, zero_v, acc_sh, cnt_sh,
             semx0, semx1, semx2, semx3, semx4, semx5,
             semi0, semi1, semi2, semi3, semi4, semi5,
             sems0, sems1, sems2, sems3, sems4, sems5):
    cid = lax.axis_index("c")
    sid = lax.axis_index("s")
    wid = cid * NS + sid
    semx = (semx0, semx1, semx2, semx3, semx4, semx5)
    semi = (semi0, semi1, semi2, semi3, semi4, semi5)
    sems = (sems0, sems1, sems2, sems3, sems4, sems5)

    zeros16 = jnp.zeros((16,), jnp.float32)
    ones16 = jnp.ones((16,), jnp.float32)
    for j in range(CHUNK // 16):
        ones_v[pl.ds(j * 16, 16)] = ones16
    for i in range(S // NS):
        for j in range(D // 16):
            zero_v[i, pl.ds(j * 16, 16)] = zeros16

    pltpu.sync_copy(zero_v, acc_sh.at[pl.ds(sid * (S // NS), S // NS)])
    pltpu.sync_copy(zero_v.at[0, pl.ds(0, S // NS)],
                    cnt_sh.at[pl.ds(sid * (S // NS), S // NS)])
    plsc.subcore_barrier()

    def chunk_row(t):
        return (wid + t * NW) * CHUNK

    def start_load(t, b):
        row = chunk_row(t)
        pltpu.async_copy(x_hbm.at[pl.ds(row, CHUNK), :], x_bufs.at[b], semx[b])
        pltpu.async_copy(batch_hbm.at[pl.ds(row, CHUNK)], idx_bufs.at[b], semi[b])

    def wait_load(t, b):
        row = chunk_row(t)
        pltpu.make_async_copy(x_hbm.at[pl.ds(row, CHUNK), :], x_bufs.at[b],
                              semx[b]).wait()
        pltpu.make_async_copy(batch_hbm.at[pl.ds(row, CHUNK)], idx_bufs.at[b],
                              semi[b]).wait()

    def start_scatter(b):
        pltpu.async_copy(x_bufs.at[b], acc_sh.at[idx_bufs.at[b]], sems[b],
                         add=True)
        pltpu.async_copy(ones_v, cnt_sh.at[idx_bufs.at[b]], sems[b], add=True)

    def wait_scatter(b):
        pltpu.make_async_copy(x_bufs.at[b], acc_sh.at[idx_bufs.at[b]],
                              sems[b]).wait()
        pltpu.make_async_copy(ones_v, cnt_sh.at[idx_bufs.at[b]],
                              sems[b]).wait()

    start_load(0, 0)
    start_load(1, 1)

    def body(tt, carry):
        for u in range(NB):
            t = tt * NB + u
            b = u
            wait_load(t, b)
            start_scatter(b)
            tf = t + 2
            bf = (u + 2) % NB

            @pl.when(tf < NBASE)
            def _():
                @pl.when(tf >= NB)
                def _():
                    wait_scatter(bf)
                start_load(tf, bf)
        return carry

    lax.fori_loop(0, NBASE // NB, body, 0)

    @pl.when(wid < NEXTRA)
    def _():
        wait_scatter(0)
        row = (NBASE * NW + wid) * CHUNK
        pltpu.async_copy(x_hbm.at[pl.ds(row, CHUNK), :], x_bufs.at[0], semx[0])
        pltpu.async_copy(batch_hbm.at[pl.ds(row, CHUNK)], idx_bufs.at[0],
                         semi[0])
        pltpu.make_async_copy(x_hbm.at[pl.ds(row, CHUNK), :], x_bufs.at[0],
                              semx[0]).wait()
        pltpu.make_async_copy(batch_hbm.at[pl.ds(row, CHUNK)], idx_bufs.at[0],
                              semi[0]).wait()
        start_scatter(0)

    for b in (1, 2, 3, 4, 5, 0):
        wait_scatter(b)

    plsc.subcore_barrier()

    r0 = sid * (S // NS)
    pltpu.sync_copy(acc_sh.at[pl.ds(r0, S // NS)],
                    sums_hbm.at[cid, pl.ds(r0, S // NS)])
    cnt_v = ones_v.at[pl.ds(0, S // NS)]
    pltpu.sync_copy(cnt_sh.at[pl.ds(r0, S // NS)], cnt_v)
    pltpu.sync_copy(cnt_v, cnts_hbm.at[cid, pl.ds(r0, S // NS)])


_sc_segment_sum = functools.partial(
    pl.kernel,
    out_type=[
        jax.ShapeDtypeStruct((NC, S, D), jnp.float32),
        jax.ShapeDtypeStruct((NC, S), jnp.float32),
    ],
    mesh=plsc.VectorSubcoreMesh(core_axis_name="c", subcore_axis_name="s"),
    scratch_types=[
        pltpu.VMEM((NB, CHUNK, D), jnp.float32),
        pltpu.VMEM((NB, CHUNK), jnp.int32),
        pltpu.VMEM((CHUNK,), jnp.float32),
        pltpu.VMEM((S // NS, D), jnp.float32),
        pltpu.VMEM_SHARED((S, D), jnp.float32),
        pltpu.VMEM_SHARED((S,), jnp.float32),
    ] + [pltpu.SemaphoreType.DMA] * 18,
)(_sc_body)


def _finalize_body(s_ref, c_ref, o_ref):
    s = s_ref[0] + s_ref[1]
    c = jnp.maximum(c_ref[0] + c_ref[1], 1.0)
    o_ref[...] = s / c


_finalize = pl.pallas_call(
    _finalize_body,
    out_shape=jax.ShapeDtypeStruct((S, D), jnp.float32),
)


@jax.jit
def kernel(x, batch):
    sums, cnts = _sc_segment_sum(x, batch.astype(jnp.int32))
    return _finalize(sums, cnts.reshape(NC, S, 1))
